# baseline (speedup 1.0000x reference)
import functools

import jax
import jax.numpy as jnp
from jax import lax
from jax.experimental import pallas as pl
from jax.experimental.pallas import tpu as pltpu
from jax.experimental.pallas import tpu_sc as plsc


def kernel(features, nodes):
    V, D = features.shape
    (B,) = nodes.shape
    assert D == 64 and V % 2 == 0

    info = plsc.get_sparse_core_info()
    nc, ns, L = info.num_cores, info.num_subcores, info.num_lanes
    nw = nc * ns
    assert B % nw == 0
    b_per_w = B // nw

    C = 512
    assert b_per_w % C == 0
    n_chunks = b_per_w // C

    feat2 = features.reshape(V // 2, 2 * D)

    mesh = plsc.VectorSubcoreMesh(core_axis_name="c", subcore_axis_name="s")

    @functools.partial(
        pl.kernel,
        mesh=mesh,
        compiler_params=pltpu.CompilerParams(needs_layout_passes=False),
        out_type=jax.ShapeDtypeStruct((B * D,), features.dtype),
        scratch_types=[
            pltpu.VMEM((C,), jnp.int32),
            pltpu.VMEM((C,), jnp.int32),
            pltpu.VMEM((C, 2 * D), features.dtype),
            pltpu.VMEM((C * D,), features.dtype),
            pltpu.SemaphoreType.DMA,
        ],
    )
    def gather_kernel(table_hbm, idx_hbm, out_hbm, idx_v, row_v, rows_v,
                      out_v, sem):
        wid = lax.axis_index("s") * nc + lax.axis_index("c")
        base = wid * b_per_w

        def chunk_body(g, carry):
            off = base + g * C
            pltpu.sync_copy(idx_hbm.at[pl.ds(off, C)], idx_v)

            def rid_body(b, carry):
                t = idx_v[pl.ds(b * L, L)]
                row_v[pl.ds(b * L, L)] = t >> 1
                return carry

            lax.fori_loop(0, C // L, rid_body, 0)

            pltpu.async_copy(table_hbm.at[row_v], rows_v, sem).wait()

            def sel_body(b, carry):
                i_vec = lax.iota(jnp.int32, L) + b * L
                h = idx_v[pl.ds(b * L, L)] & 1
                col0 = h * D
                dst0 = i_vec * D
                for k in range(D):
                    vals = plsc.load_gather(rows_v, [i_vec, col0 + k])
                    plsc.store_scatter(out_v, [dst0 + k], vals)
                return carry

            lax.fori_loop(0, C // L, sel_body, 0)

            pltpu.sync_copy(out_v, out_hbm.at[pl.ds(off * D, C * D)])
            return carry

        lax.fori_loop(0, n_chunks, chunk_body, 0)

    out_flat = gather_kernel(feat2, nodes.astype(jnp.int32))
    return out_flat.reshape(B, D)


# speedup vs baseline: 2.9548x; 2.9548x over previous
import functools

import jax
import jax.numpy as jnp
from jax import lax
from jax.experimental import pallas as pl
from jax.experimental.pallas import tpu as pltpu
from jax.experimental.pallas import tpu_sc as plsc


def kernel(features, nodes):
    V, D = features.shape
    (B,) = nodes.shape

    info = plsc.get_sparse_core_info()
    nc, ns = info.num_cores, info.num_subcores
    nw = nc * ns
    assert B % nw == 0
    b_per_w = B // nw

    chunk = 512
    assert b_per_w % chunk == 0
    n_chunks = b_per_w // chunk

    mesh = plsc.VectorSubcoreMesh(core_axis_name="c", subcore_axis_name="s")

    @functools.partial(
        pl.kernel,
        mesh=mesh,
        out_type=jax.ShapeDtypeStruct((B, D), features.dtype),
        scratch_types=[
            pltpu.VMEM((chunk,), jnp.int32),
            pltpu.VMEM((chunk, D), features.dtype),
            pltpu.SemaphoreType.DMA,
            pltpu.SemaphoreType.DMA,
        ],
    )
    def gather_kernel(table_hbm, idx_hbm, out_hbm, idx_s, rows_v, isem, sem):
        wid = lax.axis_index("s") * nc + lax.axis_index("c")
        base = wid * b_per_w

        def body(g, carry):
            off = base + g * chunk
            pltpu.async_copy(idx_hbm.at[pl.ds(off, chunk)], idx_s, isem).wait()

            def issue(b, carry):
                idx16 = idx_s[pl.ds(b * 16, 16)]
                for k in range(16):
                    r = idx16[k]
                    pltpu.async_copy(
                        table_hbm.at[pl.ds(r, 1)],
                        rows_v.at[pl.ds(b * 16 + k, 1)],
                        sem,
                    )
                return carry

            lax.fori_loop(0, chunk // 16, issue, 0)
            pltpu.make_async_copy(
                table_hbm.at[pl.ds(0, chunk)], rows_v, sem
            ).wait()
            pltpu.sync_copy(rows_v, out_hbm.at[pl.ds(off, chunk)])
            return carry

        lax.fori_loop(0, n_chunks, body, 0)

    return gather_kernel(features, nodes.astype(jnp.int32))


# speedup vs baseline: 3.0615x; 1.0361x over previous
import functools

import jax
import jax.numpy as jnp
from jax import lax
from jax.experimental import pallas as pl
from jax.experimental.pallas import tpu as pltpu
from jax.experimental.pallas import tpu_sc as plsc


def kernel(features, nodes):
    V, D = features.shape
    (B,) = nodes.shape

    info = plsc.get_sparse_core_info()
    nc, ns = info.num_cores, info.num_subcores
    nw = nc * ns
    assert B % nw == 0
    b_per_w = B // nw

    chunk = 416
    n_chunks = b_per_w // chunk
    assert b_per_w % chunk == 0 and n_chunks % 2 == 0 and n_chunks >= 4

    mesh = plsc.VectorSubcoreMesh(core_axis_name="c", subcore_axis_name="s")

    @functools.partial(
        pl.kernel,
        mesh=mesh,
        out_type=jax.ShapeDtypeStruct((B, D), features.dtype),
        scratch_types=[
            pltpu.VMEM((chunk,), jnp.int32),
            pltpu.VMEM((chunk,), jnp.int32),
            pltpu.VMEM((chunk, D), features.dtype),
            pltpu.VMEM((chunk, D), features.dtype),
            pltpu.SemaphoreType.DMA,
            pltpu.SemaphoreType.DMA,
            pltpu.SemaphoreType.DMA,
        ],
    )
    def gather_kernel(table_hbm, idx_hbm, out_hbm, idx_a, idx_b, rows_a,
                      rows_b, isem, gsem_a, gsem_b):
        wid = lax.axis_index("s") * nc + lax.axis_index("c")
        base = wid * b_per_w

        def fetch(idx_s, rows_v, gsem, g):
            off = base + g * chunk
            pltpu.async_copy(idx_hbm.at[pl.ds(off, chunk)], idx_s,
                             isem).wait()

            def issue(b, carry):
                idx16 = idx_s[pl.ds(b * 16, 16)]
                for k in range(16):
                    r = idx16[k]
                    pltpu.async_copy(
                        table_hbm.at[pl.ds(r, 1)],
                        rows_v.at[pl.ds(b * 16 + k, 1)],
                        gsem,
                    )
                return carry

            lax.fori_loop(0, chunk // 16, issue, 0)

        def finish(rows_v, gsem, g):
            pltpu.make_async_copy(table_hbm.at[pl.ds(0, chunk)], rows_v,
                                  gsem).wait()
            off = base + g * chunk
            pltpu.sync_copy(rows_v, out_hbm.at[pl.ds(off, chunk)])

        fetch(idx_a, rows_a, gsem_a, 0)

        def body(t, carry):
            g = 2 * t
            fetch(idx_b, rows_b, gsem_b, g + 1)
            finish(rows_a, gsem_a, g)
            fetch(idx_a, rows_a, gsem_a, g + 2)
            finish(rows_b, gsem_b, g + 1)
            return carry

        lax.fori_loop(0, (n_chunks - 2) // 2, body, 0)
        fetch(idx_b, rows_b, gsem_b, n_chunks - 1)
        finish(rows_a, gsem_a, n_chunks - 2)
        finish(rows_b, gsem_b, n_chunks - 1)

    return gather_kernel(features, nodes.astype(jnp.int32))


# speedup vs baseline: 4.1485x; 1.3550x over previous
import functools

import jax
import jax.numpy as jnp
from jax import lax
from jax.experimental import pallas as pl
from jax.experimental.pallas import tpu as pltpu
from jax.experimental.pallas import tpu_sc as plsc


def kernel(features, nodes):
    V, D = features.shape
    (B,) = nodes.shape
    nodes = nodes.astype(jnp.int32)

    info = plsc.get_sparse_core_info()
    nc, ns = info.num_cores, info.num_subcores
    nw = nc * ns
    assert B % nw == 0
    b_per_w = B // nw

    chunk = 416
    n_chunks = b_per_w // chunk
    assert b_per_w % chunk == 0 and n_chunks % 2 == 0 and n_chunks >= 4
    assert chunk % 16 == 0

    mesh = plsc.VectorSubcoreMesh(core_axis_name="c", subcore_axis_name="s")

    @functools.partial(
        pl.kernel,
        mesh=mesh,
        out_type=jax.ShapeDtypeStruct((1, B, D), features.dtype),
        scratch_types=[
            pltpu.VMEM((chunk,), jnp.int32),
            pltpu.VMEM((chunk,), jnp.int32),
            pltpu.VMEM((chunk, D), features.dtype),
            pltpu.VMEM((chunk, D), features.dtype),
            pltpu.SemaphoreType.DMA,
            pltpu.SemaphoreType.DMA,
            pltpu.SemaphoreType.DMA,
        ],
    )
    def gather_kernel(table_hbm, idx_hbm, out_hbm, idx_a, idx_b, rows_a,
                      rows_b, isem, gsem_a, gsem_b):
        wid = lax.axis_index("s") * nc + lax.axis_index("c")
        base = wid * b_per_w

        def fetch(idx_s, rows_v, gsem, g):
            off = base + g * chunk
            pltpu.async_copy(idx_hbm.at[pl.ds(off, chunk)], idx_s,
                             isem).wait()

            def issue(b, carry):
                idx16 = idx_s[pl.ds(b * 16, 16)]
                for k in range(16):
                    r = idx16[k]
                    pltpu.async_copy(
                        table_hbm.at[0, pl.ds(r, 1)],
                        rows_v.at[pl.ds(b * 16 + k, 1)],
                        gsem,
                    )
                return carry

            lax.fori_loop(0, chunk // 16, issue, 0)

        def finish(rows_v, gsem, g):
            pltpu.make_async_copy(table_hbm.at[0, pl.ds(0, chunk)], rows_v,
                                  gsem).wait()
            off = base + g * chunk
            pltpu.sync_copy(rows_v, out_hbm.at[0, pl.ds(off, chunk)])

        fetch(idx_a, rows_a, gsem_a, 0)

        def body(t, carry):
            g = 2 * t
            fetch(idx_b, rows_b, gsem_b, g + 1)
            finish(rows_a, gsem_a, g)
            fetch(idx_a, rows_a, gsem_a, g + 2)
            finish(rows_b, gsem_b, g + 1)
            return carry

        lax.fori_loop(0, (n_chunks - 2) // 2, body, 0)
        fetch(idx_b, rows_b, gsem_b, n_chunks - 1)
        finish(rows_a, gsem_a, n_chunks - 2)
        finish(rows_b, gsem_b, n_chunks - 1)

    out3 = gather_kernel(features.reshape(1, V, D), nodes)
    return out3.reshape(B, D)


# speedup vs baseline: 4.1559x; 1.0018x over previous
import functools

import jax
import jax.numpy as jnp
from jax import lax
from jax.experimental import pallas as pl
from jax.experimental.pallas import tpu as pltpu
from jax.experimental.pallas import tpu_sc as plsc


def kernel(features, nodes):
    V, D = features.shape
    (B,) = nodes.shape
    nodes = nodes.astype(jnp.int32)

    info = plsc.get_sparse_core_info()
    nc, ns = info.num_cores, info.num_subcores
    nw = nc * ns
    assert B % nw == 0
    b_per_w = B // nw

    chunk = 416
    n_chunks = b_per_w // chunk
    assert b_per_w % chunk == 0 and n_chunks % 2 == 0 and n_chunks >= 4
    assert chunk % 16 == 0

    mesh = plsc.VectorSubcoreMesh(core_axis_name="c", subcore_axis_name="s")

    @functools.partial(
        pl.kernel,
        mesh=mesh,
        out_type=jax.ShapeDtypeStruct((1, B, D), features.dtype),
        scratch_types=[
            pltpu.VMEM((chunk,), jnp.int32),
            pltpu.VMEM((chunk,), jnp.int32),
            pltpu.VMEM((chunk, D), features.dtype),
            pltpu.VMEM((chunk, D), features.dtype),
            pltpu.SemaphoreType.DMA,
            pltpu.SemaphoreType.DMA,
            pltpu.SemaphoreType.DMA,
        ],
    )
    def gather_kernel(table_hbm, idx_hbm, out_hbm, idx_a, idx_b, rows_a,
                      rows_b, isem, gsem_a, gsem_b):
        wid = lax.axis_index("s") * nc + lax.axis_index("c")
        base = wid * b_per_w

        def fetch(idx_s, rows_v, gsem, g):
            off = base + g * chunk
            pltpu.async_copy(idx_hbm.at[pl.ds(off, chunk)], idx_s,
                             isem).wait()

            def issue(b, carry):
                idx16 = idx_s[pl.ds(b * 16, 16)]
                rs = [idx16[k] for k in range(16)]
                for k in range(16):
                    pltpu.async_copy(
                        table_hbm.at[0, pl.ds(rs[k], 1)],
                        rows_v.at[pl.ds(b * 16 + k, 1)],
                        gsem,
                    )
                return carry

            lax.fori_loop(0, chunk // 16, issue, 0)

        def finish(rows_v, gsem, g):
            pltpu.make_async_copy(table_hbm.at[0, pl.ds(0, chunk)], rows_v,
                                  gsem).wait()
            off = base + g * chunk
            pltpu.sync_copy(rows_v, out_hbm.at[0, pl.ds(off, chunk)])

        fetch(idx_a, rows_a, gsem_a, 0)

        def body(t, carry):
            g = 2 * t
            fetch(idx_b, rows_b, gsem_b, g + 1)
            finish(rows_a, gsem_a, g)
            fetch(idx_a, rows_a, gsem_a, g + 2)
            finish(rows_b, gsem_b, g + 1)
            return carry

        lax.fori_loop(0, (n_chunks - 2) // 2, body, 0)
        fetch(idx_b, rows_b, gsem_b, n_chunks - 1)
        finish(rows_a, gsem_a, n_chunks - 2)
        finish(rows_b, gsem_b, n_chunks - 1)

    out3 = gather_kernel(features.reshape(1, V, D), nodes)
    return out3.reshape(B, D)
